# baseline (speedup 1.0000x reference)
import functools

import jax
import jax.numpy as jnp
from jax import lax
from jax.experimental import pallas as pl
from jax.experimental.pallas import tpu as pltpu
from jax.experimental.pallas import tpu_sc as plsc

NW = 32
SPW = 512
NT = 50
CB = 256
NB = SPW // CB
K = NT * NB


@functools.cache
def _build(S, T, V, D):
    mesh = plsc.VectorSubcoreMesh(core_axis_name="c", subcore_axis_name="s")

    @functools.partial(
        pl.kernel,
        out_type=jax.ShapeDtypeStruct((T, D // 8, S // 128, 8, 128), jnp.float32),
        mesh=mesh,
        scratch_types=[
            pltpu.VMEM((NT, SPW), jnp.int32),
            pltpu.VMEM((CB, 32), jnp.float32),
            pltpu.VMEM((CB, 32), jnp.float32),
            pltpu.VMEM((4, 2, 8, 128), jnp.float32),
            pltpu.VMEM((4, 2, 8, 128), jnp.float32),
            pltpu.SemaphoreType.DMA((2,)),
            pltpu.SemaphoreType.DMA((2,)),
        ],
        compiler_params=pltpu.CompilerParams(
            use_tc_tiling_on_sc=False, needs_layout_passes=False
        ),
    )
    def k(tbl, iT, y5, idxv, rows0, rows1, tr0, tr1, gsem, wsem):
        wid = lax.axis_index("s") * 2 + lax.axis_index("c")
        s0 = wid * SPW
        pltpu.sync_copy(iT.at[:, pl.ds(s0, SPW)], idxv)
        rows = (rows0, rows1)
        tr = (tr0, tr1)
        iotas = [lax.iota(jnp.int32, 16) + (j * 16) for j in range(16)]

        def fire_gather(kk, p):
            t = kk // NB
            b = kk % NB
            pltpu.async_copy(
                tbl.at[idxv.at[t, pl.ds(b * CB, CB)]], rows[p], gsem.at[p]
            )

        def wait_gather(p):
            pltpu.make_async_copy(tbl.at[pl.ds(0, CB)], rows[p], gsem.at[p]).wait()

        def transpose(p):
            for dt in range(4):
                for ds_ in range(8):
                    d = dt * 8 + ds_
                    col = jnp.full((16,), d, jnp.int32)
                    vs = [
                        plsc.load_gather(rows[p], [iotas[j], col]) for j in range(16)
                    ]
                    for j in range(16):
                        tr[p][dt, j // 8, ds_, pl.ds((j % 8) * 16, 16)] = vs[j]

        def fire_out(kk, p):
            t = kk // NB
            b = kk % NB
            stg = wid * 4 + b * 2
            pltpu.async_copy(tr[p], y5.at[t, :, pl.ds(stg, 2)], wsem.at[p])

        def wait_out(p):
            pltpu.make_async_copy(y5.at[0, :, pl.ds(0, 2)], tr[p], wsem.at[p]).wait()

        fire_gather(0, 0)
        fire_gather(1, 1)

        @pl.loop(0, K // 2)
        def _i(i):
            for p in range(2):
                kk = 2 * i + p
                wait_gather(p)

                @pl.when(i > 0)
                def _():
                    wait_out(p)

                transpose(p)

                @pl.when(kk + 2 < K)
                def _():
                    fire_gather(kk + 2, p)

                fire_out(kk, p)

        wait_out(0)
        wait_out(1)

    return k


def kernel(token_ids, embedding_matrix):
    S, T = token_ids.shape
    V, D = embedding_matrix.shape
    iT = token_ids.T.astype(jnp.int32)
    y5 = _build(S, T, V, D)(embedding_matrix, iT)
    return y5.transpose(2, 4, 0, 1, 3).reshape(S, T, D)


# speedup vs baseline: 1.4694x; 1.4694x over previous
import functools

import jax
import jax.numpy as jnp
from jax import lax
from jax.experimental import pallas as pl
from jax.experimental.pallas import tpu as pltpu
from jax.experimental.pallas import tpu_sc as plsc

NW = 32
SPW = 512
NT = 50
NB = SPW // 128
K = NT * NB
P = 129


@functools.cache
def _build(S, T, V, D):
    mesh = plsc.VectorSubcoreMesh(core_axis_name="c", subcore_axis_name="s")

    @functools.partial(
        pl.kernel,
        out_type=jax.ShapeDtypeStruct((T, D // 8, S // 128, 8, 128), jnp.float32),
        mesh=mesh,
        scratch_types=[
            pltpu.VMEM((NT, SPW), jnp.int32),
            pltpu.VMEM((128, 32), jnp.float32),
            pltpu.VMEM((128, 32), jnp.float32),
            pltpu.VMEM((4, 8, P), jnp.float32),
            pltpu.VMEM((4, 8, P), jnp.float32),
            pltpu.SemaphoreType.DMA((2,)),
            pltpu.SemaphoreType.DMA((2,)),
        ],
        compiler_params=pltpu.CompilerParams(
            use_tc_tiling_on_sc=False, needs_layout_passes=False
        ),
    )
    def k(tbl, iT, y5, idxv, rows0, rows1, tr0, tr1, gsem, wsem):
        wid = lax.axis_index("s") * 2 + lax.axis_index("c")
        s0 = wid * SPW
        pltpu.sync_copy(iT.at[:, pl.ds(s0, SPW)], idxv)
        rows = (rows0, rows1)
        tr = (tr0, tr1)
        lane = lax.iota(jnp.int32, 16)
        dt_lo = lane // 8
        dt_hi = dt_lo + 2
        ds_v = lane % 8

        def fire_gather(kk, p):
            t = kk // NB
            b = kk % NB
            pltpu.async_copy(
                tbl.at[idxv.at[t, pl.ds(b * 128, 128)]], rows[p], gsem.at[p]
            )

        def wait_gather(p):
            pltpu.make_async_copy(tbl.at[pl.ds(0, 128)], rows[p], gsem.at[p]).wait()

        def transpose(p):
            for sl in range(128):
                slv = jnp.full((16,), sl, jnp.int32)
                v0 = rows[p][sl, pl.ds(0, 16)]
                v1 = rows[p][sl, pl.ds(16, 16)]
                plsc.store_scatter(tr[p], [dt_lo, ds_v, slv], v0)
                plsc.store_scatter(tr[p], [dt_hi, ds_v, slv], v1)

        def fire_out(kk, p):
            t = kk // NB
            b = kk % NB
            stg = wid * NB + b
            pltpu.async_copy(
                tr[p].at[:, :, pl.ds(0, 128)], y5.at[t, :, stg], wsem.at[p]
            )

        def wait_out(p):
            pltpu.make_async_copy(
                y5.at[0, :, 0], tr[p].at[:, :, pl.ds(0, 128)], wsem.at[p]
            ).wait()

        fire_gather(0, 0)
        fire_gather(1, 1)

        @pl.loop(0, K // 2)
        def _i(i):
            for p in range(2):
                kk = 2 * i + p
                wait_gather(p)

                @pl.when(i > 0)
                def _():
                    wait_out(p)

                transpose(p)

                @pl.when(kk + 2 < K)
                def _():
                    fire_gather(kk + 2, p)

                fire_out(kk, p)

        wait_out(0)
        wait_out(1)

    return k


def kernel(token_ids, embedding_matrix):
    S, T = token_ids.shape
    V, D = embedding_matrix.shape
    iT = token_ids.T.astype(jnp.int32)
    y5 = _build(S, T, V, D)(embedding_matrix, iT)
    return y5.transpose(2, 4, 0, 1, 3).reshape(S, T, D)
